# Initial kernel scaffold; baseline (speedup 1.0000x reference)
import jax
import jax.numpy as jnp
from jax.experimental import pallas as pl


def kernel(xt, qt, ht, qt_kc, emb_x_table, emb_c_table, se_w1, se_w2, fs_w1, fs_b1, fs_w2, fs_b2, fn_w1, fn_b1, fn_w2, fn_b2, ea_w, ea_we, ea_be, ea_wa, ea_ba, gru_wih, gru_bih, gru_whh, gru_bhh, wp, bp, graphs):
    raise NotImplementedError("write your pallas kernel here")


# speedup vs baseline: 2.9623x; 2.9623x over previous
import jax
import jax.numpy as jnp
from jax.experimental import pallas as pl
from jax.experimental.pallas import tpu as pltpu

_B, _C, _H, _E = 256, 1024, 32, 32
_D = _H + _E
_ET = 2
_BT = 8


def _mm(x, w):
    a, b, k = x.shape
    y = jnp.dot(x.reshape(a * b, k), w, preferred_element_type=jnp.float32)
    return y.reshape(a, b, -1)


def _gkt_kernel(qt_s,
                xt_ref, qt2_ref, ht_ref, qt_kc_ref, x_emb_ref, emb_c_ref,
                se_w1_ref, se_w2_ref,
                fs_w1_ref, fs_b1_ref, fs_w2_ref, fs_b2_ref,
                fn_w1_ref, fn_b1_ref, fn_w2_ref, fn_b2_ref,
                ea_w_ref, ea_we_ref, ea_be_ref, ea_wa_ref, ea_ba_ref,
                gwih_ref, gbih_ref, gwhh_ref, gbhh_ref,
                wp_ref, bp_ref, graphs_ref, out_ref):
    i = pl.program_id(0)
    base = i * _BT

    ht = ht_ref[...]
    xt = xt_ref[...]
    qt_col = qt2_ref[...]

    x_emb = x_emb_ref[...]
    s_row = jnp.mean(x_emb, axis=1)[None, :]
    h1 = jnp.maximum(jnp.dot(s_row, se_w1_ref[...],
                             preferred_element_type=jnp.float32), 0.0)
    scale_row = jax.nn.sigmoid(jnp.dot(h1, se_w2_ref[...],
                                       preferred_element_type=jnp.float32))

    rows = []
    for j in range(_BT):
        q = qt_s[base + j]
        rows.append(qt_kc_ref[pl.ds(q, 1), :])
    mf = jnp.concatenate(rows, axis=0)

    res_emb = jnp.dot((mf * scale_row) * xt, x_emb,
                      preferred_element_type=jnp.float32)

    ceqf = (jax.lax.broadcasted_iota(jnp.int32, (_BT, _C), 1)
            == qt_col).astype(jnp.float32)
    ceq3 = ceqf[:, :, None]
    mf3 = mf[:, :, None]
    e0 = emb_c_ref[0:1, :][None]
    e1 = emb_c_ref[1:2, :][None]
    qc = e0 + mf3 * (e1 - e0)
    qc = qc + ceq3 * (res_emb[:, None, :] - qc)
    tmp_ht = jnp.concatenate([ht, qc], axis=-1)

    self_ht = jnp.sum(tmp_ht * ceq3, axis=1)
    sf1 = jnp.maximum(jnp.dot(self_ht, fs_w1_ref[...],
                              preferred_element_type=jnp.float32)
                      + fs_b1_ref[...], 0.0)
    self_feat = (jnp.dot(sf1, fs_w2_ref[...],
                         preferred_element_type=jnp.float32)
                 + fs_b2_ref[...])

    denom = jnp.maximum(jnp.sum(mf, axis=1, keepdims=True), 1.0)
    mfn = mf / denom

    nf = jnp.zeros((_BT, _C, _H), jnp.float32)
    for k in range(_ET):
        adj = jnp.dot(mfn, graphs_ref[k],
                      preferred_element_type=jnp.float32)
        w1k = fn_w1_ref[k]
        self_part = (jnp.dot(self_ht, w1k[:_D, :],
                             preferred_element_type=jnp.float32)
                     + fn_b1_ref[k][None, :])
        h1k = jnp.maximum(_mm(tmp_ht, w1k[_D:, :])
                          + self_part[:, None, :], 0.0)
        ok = _mm(h1k, fn_w2_ref[k]) + fn_b2_ref[k][None, None, :]
        nf = nf + adj[:, :, None] * ok

    m_next = nf + ceq3 * (self_feat[:, None, :] - nf)

    eg = jax.nn.sigmoid(_mm(m_next, ea_we_ref[...]) + ea_be_ref[...][None])
    ag = jnp.tanh(_mm(m_next, ea_wa_ref[...]) + ea_ba_ref[...][None])
    w3 = ea_w_ref[...][None]
    m2 = m_next - w3 * eg * m_next + w3 * ag

    gi = _mm(m2, gwih_ref[...]) + gbih_ref[...][None]
    gh = _mm(ht, gwhh_ref[...]) + gbhh_ref[...][None]
    r = jax.nn.sigmoid(gi[..., :_H] + gh[..., :_H])
    z = jax.nn.sigmoid(gi[..., _H:2 * _H] + gh[..., _H:2 * _H])
    n = jnp.tanh(gi[..., 2 * _H:] + r * gh[..., 2 * _H:])
    h_next = (1.0 - z) * n + z * ht

    wp3 = wp_ref[...][None]
    yt = jax.nn.sigmoid(jnp.sum(h_next * wp3, axis=-1) + bp_ref[0, 0])
    out_ref[...] = yt


def kernel(xt, qt, ht, qt_kc, emb_x_table, emb_c_table, se_w1, se_w2,
           fs_w1, fs_b1, fs_w2, fs_b2, fn_w1, fn_b1, fn_w2, fn_b2,
           ea_w, ea_we, ea_be, ea_wa, ea_ba,
           gru_wih, gru_bih, gru_whh, gru_bhh, wp, bp, graphs):
    x_emb = emb_x_table[:_C]

    def full(a):
        nd = a.ndim
        return pl.BlockSpec(a.shape, lambda i, q, _n=nd: (0,) * _n)

    operands = (
        xt.reshape(_B, 1),
        qt.reshape(_B, 1),
        ht,
        qt_kc,
        x_emb,
        emb_c_table,
        se_w1, se_w2,
        fs_w1, fs_b1.reshape(1, _H), fs_w2, fs_b2.reshape(1, _H),
        fn_w1, fn_b1, fn_w2, fn_b2,
        ea_w.reshape(_C, 1), ea_we, ea_be.reshape(1, _H),
        ea_wa, ea_ba.reshape(1, _H),
        gru_wih, gru_bih.reshape(1, 3 * _H),
        gru_whh, gru_bhh.reshape(1, 3 * _H),
        wp.reshape(1, _H), bp.reshape(1, 1),
        graphs,
    )

    in_specs = [
        pl.BlockSpec((_BT, 1), lambda i, q: (i, 0)),
        pl.BlockSpec((_BT, 1), lambda i, q: (i, 0)),
        pl.BlockSpec((_BT, _C, _H), lambda i, q: (i, 0, 0)),
    ] + [full(a) for a in operands[3:]]

    grid_spec = pltpu.PrefetchScalarGridSpec(
        num_scalar_prefetch=1,
        grid=(_B // _BT,),
        in_specs=in_specs,
        out_specs=pl.BlockSpec((_BT, _C), lambda i, q: (i, 0)),
    )

    return pl.pallas_call(
        _gkt_kernel,
        grid_spec=grid_spec,
        out_shape=jax.ShapeDtypeStruct((_B, _C), jnp.float32),
    )(qt, *operands)


# speedup vs baseline: 4.5584x; 1.5388x over previous
import jax
import jax.numpy as jnp
from jax.experimental import pallas as pl
from jax.experimental.pallas import tpu as pltpu

_B, _C, _H, _E = 256, 1024, 32, 32
_D = _H + _E
_ET = 2
_BT = 8


def _prologue_kernel(x_emb_ref, se_w1_ref, se_w2_ref, emb_c_ref,
                     wq_ref, sc_x_emb_ref, aux_ref):
    x = x_emb_ref[...]
    s_col = jnp.mean(x, axis=1, keepdims=True)
    s_row = jnp.transpose(s_col)
    h1 = jnp.maximum(jnp.dot(s_row, se_w1_ref[...],
                             preferred_element_type=jnp.float32), 0.0)
    scale_row = jax.nn.sigmoid(jnp.dot(h1, se_w2_ref[...],
                                       preferred_element_type=jnp.float32))
    sc_x_emb_ref[...] = x * jnp.transpose(scale_row)

    e0 = jnp.transpose(emb_c_ref[0:1, :])
    e1 = jnp.transpose(emb_c_ref[1:2, :])
    ecols = jnp.concatenate([e0, e1 - e0, e1], axis=1)
    aux_ref[...] = jnp.dot(wq_ref[...], ecols,
                           preferred_element_type=jnp.float32)


def _gkt_kernel(qt_s,
                xt_ref, ht_ref, qt_kc_ref, sc_x_emb_ref, aux_ref,
                whT_ref, wqT_ref, wselfT_ref, b1cat_ref,
                w2catT_ref, b2T_ref,
                fsw1T_ref, fsb1_ref, fsw2T_ref, fsb2_ref,
                eacatT_ref, be_ref, ba_ref, eaw_row_ref,
                wihT_ref, bih_ref, whhT_ref, bhh_ref,
                wpT_ref, bp_ref, graphs_ref, out_ref):
    i = pl.program_id(0)
    base = i * _BT

    rows = []
    for j in range(_BT):
        q = qt_s[base + j]
        rows.append(qt_kc_ref[pl.ds(q, 1), :])
    mf2 = jnp.concatenate(rows, axis=0)

    denom = jnp.maximum(jnp.sum(mf2, axis=1, keepdims=True), 1.0)
    mfn2 = mf2 * (1.0 / denom)
    adj0 = jnp.dot(mfn2, graphs_ref[0],
                   preferred_element_type=jnp.float32)
    adj1 = jnp.dot(mfn2, graphs_ref[1],
                   preferred_element_type=jnp.float32)

    res2 = jnp.dot(mf2 * xt_ref[...], sc_x_emb_ref[...],
                   preferred_element_type=jnp.float32)

    qe0 = aux_ref[:, 0:1]
    qd = aux_ref[:, 1:2]
    qe1 = aux_ref[:, 2:3]
    b1cat = b1cat_ref[...]
    lane_iota = jax.lax.broadcasted_iota(jnp.int32, (1, _C), 1)

    out_rows = []
    for j in range(_BT):
        q = qt_s[base + j]
        mf_row = rows[j]
        adj0_row = adj0[j:j + 1, :]
        adj1_row = adj1[j:j + 1, :]
        ceq_row = (lane_iota == q).astype(jnp.float32)
        htT = jnp.transpose(ht_ref[j])
        res_col = jnp.transpose(res2[j:j + 1, :])

        ht_qt = jnp.sum(htT * ceq_row, axis=1, keepdims=True)
        self_col = jnp.concatenate([ht_qt, res_col], axis=0)
        f1 = jnp.maximum(jnp.dot(fsw1T_ref[...], self_col,
                                 preferred_element_type=jnp.float32)
                         + fsb1_ref[...], 0.0)
        self_feat = (jnp.dot(fsw2T_ref[...], f1,
                             preferred_element_type=jnp.float32)
                     + fsb2_ref[...])

        qr = jnp.dot(wqT_ref[...], res_col,
                     preferred_element_type=jnp.float32)
        selfc = (jnp.dot(wselfT_ref[...], self_col,
                         preferred_element_type=jnp.float32)
                 + b1cat)
        pre = (jnp.dot(whT_ref[...], htT,
                       preferred_element_type=jnp.float32)
               + (qe0 + selfc) + qd * mf_row + (qr - qe1) * ceq_row)
        h1 = jnp.maximum(pre, 0.0)

        z = jnp.concatenate([h1[:_H] * adj0_row,
                             h1[_H:] * adj1_row], axis=0)
        nf = (jnp.dot(w2catT_ref[...], z,
                      preferred_element_type=jnp.float32)
              + b2T_ref[:, 0:1] * adj0_row + b2T_ref[:, 1:2] * adj1_row)

        m_next = nf + ceq_row * (self_feat - nf)

        ea = jnp.dot(eacatT_ref[...], m_next,
                     preferred_element_type=jnp.float32)
        eg = jax.nn.sigmoid(ea[:_H] + be_ref[...])
        ag = jnp.tanh(ea[_H:] + ba_ref[...])
        w_row = eaw_row_ref[...]
        m2 = m_next - w_row * eg * m_next + w_row * ag

        gi = (jnp.dot(wihT_ref[...], m2,
                      preferred_element_type=jnp.float32)
              + bih_ref[...])
        gh = (jnp.dot(whhT_ref[...], htT,
                      preferred_element_type=jnp.float32)
              + bhh_ref[...])
        r = jax.nn.sigmoid(gi[:_H] + gh[:_H])
        zg = jax.nn.sigmoid(gi[_H:2 * _H] + gh[_H:2 * _H])
        n = jnp.tanh(gi[2 * _H:] + r * gh[2 * _H:])
        h_next = n + zg * (htT - n)

        yt_row = jax.nn.sigmoid(jnp.dot(wpT_ref[...], h_next,
                                        preferred_element_type=jnp.float32)
                                + bp_ref[0, 0])
        out_rows.append(yt_row)

    out_ref[...] = jnp.concatenate(out_rows, axis=0)


def kernel(xt, qt, ht, qt_kc, emb_x_table, emb_c_table, se_w1, se_w2,
           fs_w1, fs_b1, fs_w2, fs_b2, fn_w1, fn_b1, fn_w2, fn_b2,
           ea_w, ea_we, ea_be, ea_wa, ea_ba,
           gru_wih, gru_bih, gru_whh, gru_bhh, wp, bp, graphs):
    f32 = jnp.float32
    x_emb = emb_x_table[:_C]

    wh_T = jnp.concatenate([fn_w1[0, _D:_D + _H].T,
                            fn_w1[1, _D:_D + _H].T], axis=0)
    wq_T = jnp.concatenate([fn_w1[0, _D + _H:].T,
                            fn_w1[1, _D + _H:].T], axis=0)
    wself_T = jnp.concatenate([fn_w1[0, :_D].T,
                               fn_w1[1, :_D].T], axis=0)
    b1cat = jnp.concatenate([fn_b1[0], fn_b1[1]]).reshape(2 * _H, 1)
    w2cat_T = jnp.concatenate([fn_w2[0].T, fn_w2[1].T], axis=1)
    b2_T = jnp.stack([fn_b2[0], fn_b2[1]], axis=1)
    eacat_T = jnp.concatenate([ea_we.T, ea_wa.T], axis=0)

    sc_x_emb, aux = pl.pallas_call(
        _prologue_kernel,
        out_shape=(jax.ShapeDtypeStruct((_C, _E), f32),
                   jax.ShapeDtypeStruct((2 * _H, 3), f32)),
    )(x_emb, se_w1, se_w2, emb_c_table, wq_T)

    operands = (
        xt.reshape(_B, 1),
        ht,
        qt_kc,
        sc_x_emb,
        aux,
        wh_T, wq_T, wself_T, b1cat,
        w2cat_T, b2_T,
        fs_w1.T, fs_b1.reshape(_H, 1), fs_w2.T, fs_b2.reshape(_H, 1),
        eacat_T, ea_be.reshape(_H, 1), ea_ba.reshape(_H, 1),
        ea_w.reshape(1, _C),
        gru_wih.T, gru_bih.reshape(3 * _H, 1),
        gru_whh.T, gru_bhh.reshape(3 * _H, 1),
        wp.reshape(1, _H), bp.reshape(1, 1),
        graphs,
    )

    def full(a):
        nd = a.ndim
        return pl.BlockSpec(a.shape, lambda i, q, _n=nd: (0,) * _n)

    in_specs = [
        pl.BlockSpec((_BT, 1), lambda i, q: (i, 0)),
        pl.BlockSpec((_BT, _C, _H), lambda i, q: (i, 0, 0)),
    ] + [full(a) for a in operands[2:]]

    grid_spec = pltpu.PrefetchScalarGridSpec(
        num_scalar_prefetch=1,
        grid=(_B // _BT,),
        in_specs=in_specs,
        out_specs=pl.BlockSpec((_BT, _C), lambda i, q: (i, 0)),
    )

    return pl.pallas_call(
        _gkt_kernel,
        grid_spec=grid_spec,
        out_shape=jax.ShapeDtypeStruct((_B, _C), f32),
    )(qt, *operands)


# speedup vs baseline: 4.7515x; 1.0423x over previous
import jax
import jax.numpy as jnp
from jax.experimental import pallas as pl
from jax.experimental.pallas import tpu as pltpu

_B, _C, _H, _E = 256, 1024, 32, 32
_D = _H + _E
_ET = 2
_BT = 16


def _sig(x):
    return 0.5 * jnp.tanh(0.5 * x) + 0.5


def _prologue_kernel(x_emb_ref, se_w1_ref, se_w2_ref, emb_c_ref,
                     wq_ref, qtf_ref, xt_ref, qt_kc_ref, graphs_ref,
                     aux_ref, mf_ref, adj0_ref, adj1_ref, res_ref):
    x = x_emb_ref[...]
    s_col = jnp.mean(x, axis=1, keepdims=True)
    s_row = jnp.transpose(s_col)
    h1 = jnp.maximum(jnp.dot(s_row, se_w1_ref[...],
                             preferred_element_type=jnp.float32), 0.0)
    scale_row = _sig(jnp.dot(h1, se_w2_ref[...],
                             preferred_element_type=jnp.float32))
    sc_x_emb = x * jnp.transpose(scale_row)

    e0 = jnp.transpose(emb_c_ref[0:1, :])
    e1 = jnp.transpose(emb_c_ref[1:2, :])
    ecols = jnp.concatenate([e0, e1 - e0, e1], axis=1)
    aux_ref[...] = jnp.dot(wq_ref[...], ecols,
                           preferred_element_type=jnp.float32)

    lane_iota = jax.lax.broadcasted_iota(
        jnp.int32, (_B, _C), 1).astype(jnp.float32)
    onehot = (lane_iota == qtf_ref[...]).astype(jnp.float32)
    mf = jnp.dot(onehot, qt_kc_ref[...],
                 preferred_element_type=jnp.float32)
    mf_ref[...] = mf

    denom = jnp.maximum(jnp.sum(mf, axis=1, keepdims=True), 1.0)
    mfn = mf * (1.0 / denom)
    adj0_ref[...] = jnp.dot(mfn, graphs_ref[0],
                            preferred_element_type=jnp.float32)
    adj1_ref[...] = jnp.dot(mfn, graphs_ref[1],
                            preferred_element_type=jnp.float32)

    res_ref[...] = jnp.dot(mf * xt_ref[...], sc_x_emb,
                           preferred_element_type=jnp.float32)


def _gkt_kernel(qt_s,
                ht_ref, mf_ref, adj0_ref, adj1_ref, res_ref,
                aux_ref,
                whtcat_ref, wqT_ref, wselfT_ref, b1cat_ref,
                w2catT_ref, b2T_ref,
                fsw1T_ref, fsb1_ref, fsw2T_ref, fsb2_ref,
                eacatT_ref, be_ref, ba_ref, eaw_row_ref,
                bih_ref, bhh_ref, wihT_ref,
                wpT_ref, bp_ref, out_ref):
    i = pl.program_id(0)
    base = i * _BT

    qe0 = aux_ref[:, 0:1]
    qd = aux_ref[:, 1:2]
    qe1 = aux_ref[:, 2:3]
    b1cat = b1cat_ref[...]
    lane_iota = jax.lax.broadcasted_iota(jnp.int32, (1, _C), 1)

    out_rows = []
    for j in range(_BT):
        q = qt_s[base + j]
        mf_row = mf_ref[j:j + 1, :]
        adj0_row = adj0_ref[j:j + 1, :]
        adj1_row = adj1_ref[j:j + 1, :]
        ceq_row = (lane_iota == q).astype(jnp.float32)
        htT = jnp.transpose(ht_ref[j])
        res_col = jnp.transpose(res_ref[j:j + 1, :])

        ht_qt = jnp.sum(htT * ceq_row, axis=1, keepdims=True)
        self_col = jnp.concatenate([ht_qt, res_col], axis=0)
        f1 = jnp.maximum(jnp.dot(fsw1T_ref[...], self_col,
                                 preferred_element_type=jnp.float32)
                         + fsb1_ref[...], 0.0)
        self_feat = (jnp.dot(fsw2T_ref[...], f1,
                             preferred_element_type=jnp.float32)
                     + fsb2_ref[...])

        htcat = jnp.dot(whtcat_ref[...], htT,
                        preferred_element_type=jnp.float32)
        gh = htcat[2 * _H:, :] + bhh_ref[...]

        qr = jnp.dot(wqT_ref[...], res_col,
                     preferred_element_type=jnp.float32)
        selfc = (jnp.dot(wselfT_ref[...], self_col,
                         preferred_element_type=jnp.float32)
                 + b1cat)
        pre = (htcat[:2 * _H, :]
               + (qe0 + selfc) + qd * mf_row + (qr - qe1) * ceq_row)
        h1 = jnp.maximum(pre, 0.0)

        z = jnp.concatenate([h1[:_H] * adj0_row,
                             h1[_H:] * adj1_row], axis=0)
        nf = (jnp.dot(w2catT_ref[...], z,
                      preferred_element_type=jnp.float32)
              + b2T_ref[:, 0:1] * adj0_row + b2T_ref[:, 1:2] * adj1_row)

        m_next = nf + ceq_row * (self_feat - nf)

        ea = jnp.dot(eacatT_ref[...], m_next,
                     preferred_element_type=jnp.float32)
        eg = _sig(ea[:_H] + be_ref[...])
        ag = jnp.tanh(ea[_H:] + ba_ref[...])
        w_row = eaw_row_ref[...]
        m2 = m_next - w_row * eg * m_next + w_row * ag

        gi = (jnp.dot(wihT_ref[...], m2,
                      preferred_element_type=jnp.float32)
              + bih_ref[...])
        r = _sig(gi[:_H] + gh[:_H])
        zg = _sig(gi[_H:2 * _H] + gh[_H:2 * _H])
        n = jnp.tanh(gi[2 * _H:] + r * gh[2 * _H:])
        htT_h = htT
        h_next = n + zg * (htT_h - n)

        yt_row = _sig(jnp.dot(wpT_ref[...], h_next,
                              preferred_element_type=jnp.float32)
                      + bp_ref[0, 0])
        out_rows.append(yt_row)

    out_ref[...] = jnp.concatenate(out_rows, axis=0)


def kernel(xt, qt, ht, qt_kc, emb_x_table, emb_c_table, se_w1, se_w2,
           fs_w1, fs_b1, fs_w2, fs_b2, fn_w1, fn_b1, fn_w2, fn_b2,
           ea_w, ea_we, ea_be, ea_wa, ea_ba,
           gru_wih, gru_bih, gru_whh, gru_bhh, wp, bp, graphs):
    f32 = jnp.float32
    x_emb = emb_x_table[:_C]

    wh_T = jnp.concatenate([fn_w1[0, _D:_D + _H].T,
                            fn_w1[1, _D:_D + _H].T], axis=0)
    wq_T = jnp.concatenate([fn_w1[0, _D + _H:].T,
                            fn_w1[1, _D + _H:].T], axis=0)
    wself_T = jnp.concatenate([fn_w1[0, :_D].T,
                               fn_w1[1, :_D].T], axis=0)
    b1cat = jnp.concatenate([fn_b1[0], fn_b1[1]]).reshape(2 * _H, 1)
    w2cat_T = jnp.concatenate([fn_w2[0].T, fn_w2[1].T], axis=1)
    b2_T = jnp.stack([fn_b2[0], fn_b2[1]], axis=1)
    eacat_T = jnp.concatenate([ea_we.T, ea_wa.T], axis=0)
    whtcat = jnp.concatenate([wh_T, gru_whh.T], axis=0)

    aux, mf, adj0, adj1, res = pl.pallas_call(
        _prologue_kernel,
        out_shape=(jax.ShapeDtypeStruct((2 * _H, 3), f32),
                   jax.ShapeDtypeStruct((_B, _C), f32),
                   jax.ShapeDtypeStruct((_B, _C), f32),
                   jax.ShapeDtypeStruct((_B, _C), f32),
                   jax.ShapeDtypeStruct((_B, _E), f32)),
    )(x_emb, se_w1, se_w2, emb_c_table, wq_T,
      qt.astype(f32).reshape(_B, 1), xt.reshape(_B, 1),
      qt_kc[:_C], graphs)

    operands = (
        ht, mf, adj0, adj1, res,
        aux,
        whtcat, wq_T, wself_T, b1cat,
        w2cat_T, b2_T,
        fs_w1.T, fs_b1.reshape(_H, 1), fs_w2.T, fs_b2.reshape(_H, 1),
        eacat_T, ea_be.reshape(_H, 1), ea_ba.reshape(_H, 1),
        ea_w.reshape(1, _C),
        gru_bih.reshape(3 * _H, 1), gru_bhh.reshape(3 * _H, 1),
        gru_wih.T,
        wp.reshape(1, _H), bp.reshape(1, 1),
    )

    def full(a):
        nd = a.ndim
        return pl.BlockSpec(a.shape, lambda i, q, _n=nd: (0,) * _n)

    in_specs = [
        pl.BlockSpec((_BT, _C, _H), lambda i, q: (i, 0, 0)),
        pl.BlockSpec((_BT, _C), lambda i, q: (i, 0)),
        pl.BlockSpec((_BT, _C), lambda i, q: (i, 0)),
        pl.BlockSpec((_BT, _C), lambda i, q: (i, 0)),
        pl.BlockSpec((_BT, _E), lambda i, q: (i, 0)),
    ] + [full(a) for a in operands[5:]]

    grid_spec = pltpu.PrefetchScalarGridSpec(
        num_scalar_prefetch=1,
        grid=(_B // _BT,),
        in_specs=in_specs,
        out_specs=pl.BlockSpec((_BT, _C), lambda i, q: (i, 0)),
    )

    return pl.pallas_call(
        _gkt_kernel,
        grid_spec=grid_spec,
        out_shape=jax.ShapeDtypeStruct((_B, _C), f32),
    )(qt, *operands)


# speedup vs baseline: 5.6723x; 1.1938x over previous
import jax
import jax.numpy as jnp
from jax.experimental import pallas as pl
from jax.experimental.pallas import tpu as pltpu

_B, _C, _H, _E = 256, 1024, 32, 32
_D = _H + _E
_ET = 2
_BT = 16


def _sig(x):
    return 0.5 * jnp.tanh(0.5 * x) + 0.5


def _prologue_kernel(x_emb_ref, se_w1_ref, se_w2_ref, emb_c_ref,
                     wq_ref, qtf_ref, xt_ref, qt_kc_ref, graphs_ref,
                     aux_ref, mf_ref, adj0_ref, adj1_ref, res_ref):
    x = x_emb_ref[...]
    s_col = jnp.mean(x, axis=1, keepdims=True)
    s_row = jnp.transpose(s_col)
    h1 = jnp.maximum(jnp.dot(s_row, se_w1_ref[...],
                             preferred_element_type=jnp.float32), 0.0)
    scale_row = _sig(jnp.dot(h1, se_w2_ref[...],
                             preferred_element_type=jnp.float32))
    sc_x_emb = x * jnp.transpose(scale_row)

    e0 = jnp.transpose(emb_c_ref[0:1, :])
    e1 = jnp.transpose(emb_c_ref[1:2, :])
    ecols = jnp.concatenate([e0, e1 - e0, e1], axis=1)
    aux_ref[...] = jnp.dot(wq_ref[...], ecols,
                           preferred_element_type=jnp.float32)

    lane_iota = jax.lax.broadcasted_iota(
        jnp.int32, (_B, _C), 1).astype(jnp.float32)
    onehot = (lane_iota == qtf_ref[...]).astype(jnp.float32)
    mf = jnp.dot(onehot, qt_kc_ref[...],
                 preferred_element_type=jnp.float32)
    mf_ref[...] = mf

    denom = jnp.maximum(jnp.sum(mf, axis=1, keepdims=True), 1.0)
    mfn = mf * (1.0 / denom)
    adj0_ref[...] = jnp.dot(mfn, graphs_ref[0],
                            preferred_element_type=jnp.float32)
    adj1_ref[...] = jnp.dot(mfn, graphs_ref[1],
                            preferred_element_type=jnp.float32)

    res_ref[...] = jnp.dot(mf * xt_ref[...], sc_x_emb,
                           preferred_element_type=jnp.float32)


def _gkt_kernel(qt_s,
                ht_ref, mf_ref, adj0_ref, adj1_ref, res_ref,
                aux_ref, whtcat_ref, bhh_ref,
                pa_w_ref, fsw2aug_ref, w2aug_ref,
                eawaug_ref, wihaug_ref, eaw_row_ref, wpaug_ref,
                out_ref):
    i = pl.program_id(0)
    base = i * _BT

    qe0 = aux_ref[:, 0:1]
    qd = aux_ref[:, 1:2]
    qe1 = aux_ref[:, 2:3]
    lane_iota = jax.lax.broadcasted_iota(jnp.int32, (1, _C), 1)
    ones_c = jnp.ones((1, _C), jnp.float32)
    ones_bt = jnp.ones((1, _BT), jnp.float32)
    w_row = eaw_row_ref[...]

    ceq_rows = []
    htq_rows = []
    for j in range(_BT):
        q = qt_s[base + j]
        ceq = (lane_iota == q).astype(jnp.float32)
        ceq_rows.append(ceq)
        htq_rows.append(jnp.dot(ceq, ht_ref[j],
                                preferred_element_type=jnp.float32))
    scols = jnp.concatenate(
        [jnp.concatenate(htq_rows, axis=0), res_ref[...]], axis=1)
    scols_aug = jnp.concatenate(
        [jnp.transpose(scols), ones_bt], axis=0)

    pa = jnp.dot(pa_w_ref[...], scols_aug,
                 preferred_element_type=jnp.float32)
    f1 = jnp.maximum(pa[:_H], 0.0)
    self_feat_all = jnp.dot(fsw2aug_ref[...],
                            jnp.concatenate([f1, ones_bt], axis=0),
                            preferred_element_type=jnp.float32)
    colones_fn = pa[_H:3 * _H] + qe0
    colceq_fn = pa[3 * _H:] - qe1
    cols_gru = jnp.concatenate(
        [bhh_ref[...], jnp.zeros((3 * _H, 2), jnp.float32)],
        axis=1)

    out_rows = []
    for j in range(_BT):
        ceq_row = ceq_rows[j]
        mf_row = mf_ref[j:j + 1, :]
        adj0_row = adj0_ref[j:j + 1, :]
        adj1_row = adj1_ref[j:j + 1, :]
        htT = jnp.transpose(ht_ref[j])

        cols_fn = jnp.concatenate(
            [colones_fn[:, j:j + 1], qd, colceq_fn[:, j:j + 1]],
            axis=1)
        w_aug = jnp.concatenate(
            [whtcat_ref[...],
             jnp.concatenate([cols_fn, cols_gru], axis=0)], axis=1)
        aug_in = jnp.concatenate(
            [htT, ones_c, mf_row, ceq_row], axis=0)
        big = jnp.dot(w_aug, aug_in,
                      preferred_element_type=jnp.float32)
        h1 = jnp.maximum(big[:2 * _H], 0.0)
        gh = big[2 * _H:]

        z = jnp.concatenate([h1[:_H] * adj0_row,
                             h1[_H:] * adj1_row,
                             adj0_row, adj1_row], axis=0)
        nf = jnp.dot(w2aug_ref[...], z,
                     preferred_element_type=jnp.float32)

        m_next = nf + ceq_row * (self_feat_all[:, j:j + 1] - nf)

        ea = jnp.dot(eawaug_ref[...],
                     jnp.concatenate([m_next, ones_c], axis=0),
                     preferred_element_type=jnp.float32)
        eg = _sig(ea[:_H])
        ag = jnp.tanh(ea[_H:])
        m2 = m_next - (w_row * eg) * m_next + w_row * ag

        gi = jnp.dot(wihaug_ref[...],
                     jnp.concatenate([m2, ones_c], axis=0),
                     preferred_element_type=jnp.float32)
        r = _sig(gi[:_H] + gh[:_H])
        zg = _sig(gi[_H:2 * _H] + gh[_H:2 * _H])
        n = jnp.tanh(gi[2 * _H:] + r * gh[2 * _H:])
        h_next = n + zg * (htT - n)

        yt_row = _sig(jnp.dot(wpaug_ref[...],
                              jnp.concatenate([h_next, ones_c], axis=0),
                              preferred_element_type=jnp.float32))
        out_rows.append(yt_row)

    out_ref[...] = jnp.concatenate(out_rows, axis=0)


def kernel(xt, qt, ht, qt_kc, emb_x_table, emb_c_table, se_w1, se_w2,
           fs_w1, fs_b1, fs_w2, fs_b2, fn_w1, fn_b1, fn_w2, fn_b2,
           ea_w, ea_we, ea_be, ea_wa, ea_ba,
           gru_wih, gru_bih, gru_whh, gru_bhh, wp, bp, graphs):
    f32 = jnp.float32
    x_emb = emb_x_table[:_C]

    wh_T = jnp.concatenate([fn_w1[0, _D:_D + _H].T,
                            fn_w1[1, _D:_D + _H].T], axis=0)
    wq_T = jnp.concatenate([fn_w1[0, _D + _H:].T,
                            fn_w1[1, _D + _H:].T], axis=0)
    wself_T = jnp.concatenate([fn_w1[0, :_D].T,
                               fn_w1[1, :_D].T], axis=0)
    b1cat = jnp.concatenate([fn_b1[0], fn_b1[1]]).reshape(2 * _H, 1)
    w2cat_T = jnp.concatenate([fn_w2[0].T, fn_w2[1].T], axis=1)
    b2_T = jnp.stack([fn_b2[0], fn_b2[1]], axis=1)
    eacat_T = jnp.concatenate([ea_we.T, ea_wa.T], axis=0)
    whtcat = jnp.concatenate([wh_T, gru_whh.T], axis=0)

    pa_w = jnp.concatenate([
        jnp.concatenate([fs_w1.T, fs_b1.reshape(_H, 1)], axis=1),
        jnp.concatenate([wself_T, b1cat], axis=1),
        jnp.concatenate([jnp.zeros((2 * _H, _H), f32), wq_T,
                         jnp.zeros((2 * _H, 1), f32)], axis=1),
    ], axis=0)
    fsw2_aug = jnp.concatenate([fs_w2.T, fs_b2.reshape(_H, 1)], axis=1)
    w2aug = jnp.concatenate([w2cat_T, b2_T], axis=1)
    ea_w_aug = jnp.concatenate(
        [eacat_T,
         jnp.concatenate([ea_be, ea_ba]).reshape(2 * _H, 1)], axis=1)
    wih_aug = jnp.concatenate([gru_wih.T, gru_bih.reshape(3 * _H, 1)],
                              axis=1)
    wp_aug = jnp.concatenate([wp.reshape(1, _H), bp.reshape(1, 1)],
                             axis=1)

    aux, mf, adj0, adj1, res = pl.pallas_call(
        _prologue_kernel,
        out_shape=(jax.ShapeDtypeStruct((2 * _H, 3), f32),
                   jax.ShapeDtypeStruct((_B, _C), f32),
                   jax.ShapeDtypeStruct((_B, _C), f32),
                   jax.ShapeDtypeStruct((_B, _C), f32),
                   jax.ShapeDtypeStruct((_B, _E), f32)),
    )(x_emb, se_w1, se_w2, emb_c_table, wq_T,
      qt.astype(f32).reshape(_B, 1), xt.reshape(_B, 1),
      qt_kc[:_C], graphs)

    operands = (
        ht, mf, adj0, adj1, res,
        aux, whtcat, gru_bhh.reshape(3 * _H, 1),
        pa_w, fsw2_aug, w2aug,
        ea_w_aug, wih_aug, ea_w.reshape(1, _C), wp_aug,
    )

    def full(a):
        nd = a.ndim
        return pl.BlockSpec(a.shape, lambda i, q, _n=nd: (0,) * _n)

    in_specs = [
        pl.BlockSpec((_BT, _C, _H), lambda i, q: (i, 0, 0)),
        pl.BlockSpec((_BT, _C), lambda i, q: (i, 0)),
        pl.BlockSpec((_BT, _C), lambda i, q: (i, 0)),
        pl.BlockSpec((_BT, _C), lambda i, q: (i, 0)),
        pl.BlockSpec((_BT, _E), lambda i, q: (i, 0)),
    ] + [full(a) for a in operands[5:]]

    grid_spec = pltpu.PrefetchScalarGridSpec(
        num_scalar_prefetch=1,
        grid=(_B // _BT,),
        in_specs=in_specs,
        out_specs=pl.BlockSpec((_BT, _C), lambda i, q: (i, 0)),
    )

    return pl.pallas_call(
        _gkt_kernel,
        grid_spec=grid_spec,
        out_shape=jax.ShapeDtypeStruct((_B, _C), f32),
    )(qt, *operands)


# speedup vs baseline: 5.7299x; 1.0102x over previous
import jax
import jax.numpy as jnp
from jax.experimental import pallas as pl
from jax.experimental.pallas import tpu as pltpu

_B, _C, _H, _E = 256, 1024, 32, 32
_D = _H + _E
_ET = 2
_BT = 32


def _sig(x):
    return 0.5 * jnp.tanh(0.5 * x) + 0.5


def _prologue_kernel(x_emb_ref, se_w1_ref, se_w2_ref, emb_c_ref,
                     wq_ref, qtf_ref, xt_ref, qt_kc_ref, graphs_ref,
                     aux_ref, mf_ref, adj0_ref, adj1_ref, res_ref):
    x = x_emb_ref[...]
    s_col = jnp.mean(x, axis=1, keepdims=True)
    s_row = jnp.transpose(s_col)
    h1 = jnp.maximum(jnp.dot(s_row, se_w1_ref[...],
                             preferred_element_type=jnp.float32), 0.0)
    scale_row = _sig(jnp.dot(h1, se_w2_ref[...],
                             preferred_element_type=jnp.float32))
    sc_x_emb = x * jnp.transpose(scale_row)

    e0 = jnp.transpose(emb_c_ref[0:1, :])
    e1 = jnp.transpose(emb_c_ref[1:2, :])
    ecols = jnp.concatenate([e0, e1 - e0, e1], axis=1)
    aux_ref[...] = jnp.dot(wq_ref[...], ecols,
                           preferred_element_type=jnp.float32)

    lane_iota = jax.lax.broadcasted_iota(
        jnp.int32, (_B, _C), 1).astype(jnp.float32)
    onehot = (lane_iota == qtf_ref[...]).astype(jnp.float32)
    mf = jnp.dot(onehot, qt_kc_ref[...],
                 preferred_element_type=jnp.float32)
    mf_ref[...] = mf

    denom = jnp.maximum(jnp.sum(mf, axis=1, keepdims=True), 1.0)
    mfn = mf * (1.0 / denom)
    adj0_ref[...] = jnp.dot(mfn, graphs_ref[0],
                            preferred_element_type=jnp.float32)
    adj1_ref[...] = jnp.dot(mfn, graphs_ref[1],
                            preferred_element_type=jnp.float32)

    res_ref[...] = jnp.dot(mf * xt_ref[...], sc_x_emb,
                           preferred_element_type=jnp.float32)


def _gkt_kernel(qt_s,
                ht_ref, mf_ref, adj0_ref, adj1_ref, res_ref,
                aux_ref, whtcat_ref, bhh_ref,
                pa_w_ref, fsw2aug_ref, w2aug_ref,
                eawaug_ref, wihaug_ref, eaw_row_ref, wpaug_ref,
                out_ref):
    i = pl.program_id(0)
    base = i * _BT

    qe0 = aux_ref[:, 0:1]
    qd = aux_ref[:, 1:2]
    qe1 = aux_ref[:, 2:3]
    lane_iota = jax.lax.broadcasted_iota(jnp.int32, (1, _C), 1)
    ones_c = jnp.ones((1, _C), jnp.float32)
    ones_bt = jnp.ones((1, _BT), jnp.float32)
    w_row = eaw_row_ref[...]

    ceq_rows = []
    htq_rows = []
    for j in range(_BT):
        q = qt_s[base + j]
        ceq = (lane_iota == q).astype(jnp.float32)
        ceq_rows.append(ceq)
        htq_rows.append(jnp.dot(ceq, ht_ref[j],
                                preferred_element_type=jnp.float32))
    scols = jnp.concatenate(
        [jnp.concatenate(htq_rows, axis=0), res_ref[...]], axis=1)
    scols_aug = jnp.concatenate(
        [jnp.transpose(scols), ones_bt], axis=0)

    pa = jnp.dot(pa_w_ref[...], scols_aug,
                 preferred_element_type=jnp.float32)
    f1 = jnp.maximum(pa[:_H], 0.0)
    self_feat_all = jnp.dot(fsw2aug_ref[...],
                            jnp.concatenate([f1, ones_bt], axis=0),
                            preferred_element_type=jnp.float32)
    colones_fn = pa[_H:3 * _H] + qe0
    colceq_fn = pa[3 * _H:] - qe1
    cols_gru = jnp.concatenate(
        [bhh_ref[...], jnp.zeros((3 * _H, 2), jnp.float32)],
        axis=1)

    out_rows = []
    for j in range(_BT):
        ceq_row = ceq_rows[j]
        mf_row = mf_ref[j:j + 1, :]
        adj0_row = adj0_ref[j:j + 1, :]
        adj1_row = adj1_ref[j:j + 1, :]
        htT = jnp.transpose(ht_ref[j])

        cols_fn = jnp.concatenate(
            [colones_fn[:, j:j + 1], qd, colceq_fn[:, j:j + 1]],
            axis=1)
        w_aug = jnp.concatenate(
            [whtcat_ref[...],
             jnp.concatenate([cols_fn, cols_gru], axis=0)], axis=1)
        aug_in = jnp.concatenate(
            [htT, ones_c, mf_row, ceq_row], axis=0)
        big = jnp.dot(w_aug, aug_in,
                      preferred_element_type=jnp.float32)
        h1 = jnp.maximum(big[:2 * _H], 0.0)
        gh = big[2 * _H:]

        z = jnp.concatenate([h1[:_H] * adj0_row,
                             h1[_H:] * adj1_row,
                             adj0_row, adj1_row], axis=0)
        nf = jnp.dot(w2aug_ref[...], z,
                     preferred_element_type=jnp.float32)

        m_next = nf + ceq_row * (self_feat_all[:, j:j + 1] - nf)

        ea = jnp.dot(eawaug_ref[...],
                     jnp.concatenate([m_next, ones_c], axis=0),
                     preferred_element_type=jnp.float32)
        eg = _sig(ea[:_H])
        ag = jnp.tanh(ea[_H:])
        m2 = m_next - (w_row * eg) * m_next + w_row * ag

        gi = jnp.dot(wihaug_ref[...],
                     jnp.concatenate([m2, ones_c], axis=0),
                     preferred_element_type=jnp.float32)
        r = _sig(gi[:_H] + gh[:_H])
        zg = _sig(gi[_H:2 * _H] + gh[_H:2 * _H])
        n = jnp.tanh(gi[2 * _H:] + r * gh[2 * _H:])
        h_next = n + zg * (htT - n)

        yt_row = _sig(jnp.dot(wpaug_ref[...],
                              jnp.concatenate([h_next, ones_c], axis=0),
                              preferred_element_type=jnp.float32))
        out_rows.append(yt_row)

    out_ref[...] = jnp.concatenate(out_rows, axis=0)


def kernel(xt, qt, ht, qt_kc, emb_x_table, emb_c_table, se_w1, se_w2,
           fs_w1, fs_b1, fs_w2, fs_b2, fn_w1, fn_b1, fn_w2, fn_b2,
           ea_w, ea_we, ea_be, ea_wa, ea_ba,
           gru_wih, gru_bih, gru_whh, gru_bhh, wp, bp, graphs):
    f32 = jnp.float32
    x_emb = emb_x_table[:_C]

    wh_T = jnp.concatenate([fn_w1[0, _D:_D + _H].T,
                            fn_w1[1, _D:_D + _H].T], axis=0)
    wq_T = jnp.concatenate([fn_w1[0, _D + _H:].T,
                            fn_w1[1, _D + _H:].T], axis=0)
    wself_T = jnp.concatenate([fn_w1[0, :_D].T,
                               fn_w1[1, :_D].T], axis=0)
    b1cat = jnp.concatenate([fn_b1[0], fn_b1[1]]).reshape(2 * _H, 1)
    w2cat_T = jnp.concatenate([fn_w2[0].T, fn_w2[1].T], axis=1)
    b2_T = jnp.stack([fn_b2[0], fn_b2[1]], axis=1)
    eacat_T = jnp.concatenate([ea_we.T, ea_wa.T], axis=0)
    whtcat = jnp.concatenate([wh_T, gru_whh.T], axis=0)

    pa_w = jnp.concatenate([
        jnp.concatenate([fs_w1.T, fs_b1.reshape(_H, 1)], axis=1),
        jnp.concatenate([wself_T, b1cat], axis=1),
        jnp.concatenate([jnp.zeros((2 * _H, _H), f32), wq_T,
                         jnp.zeros((2 * _H, 1), f32)], axis=1),
    ], axis=0)
    fsw2_aug = jnp.concatenate([fs_w2.T, fs_b2.reshape(_H, 1)], axis=1)
    w2aug = jnp.concatenate([w2cat_T, b2_T], axis=1)
    ea_w_aug = jnp.concatenate(
        [eacat_T,
         jnp.concatenate([ea_be, ea_ba]).reshape(2 * _H, 1)], axis=1)
    wih_aug = jnp.concatenate([gru_wih.T, gru_bih.reshape(3 * _H, 1)],
                              axis=1)
    wp_aug = jnp.concatenate([wp.reshape(1, _H), bp.reshape(1, 1)],
                             axis=1)

    aux, mf, adj0, adj1, res = pl.pallas_call(
        _prologue_kernel,
        out_shape=(jax.ShapeDtypeStruct((2 * _H, 3), f32),
                   jax.ShapeDtypeStruct((_B, _C), f32),
                   jax.ShapeDtypeStruct((_B, _C), f32),
                   jax.ShapeDtypeStruct((_B, _C), f32),
                   jax.ShapeDtypeStruct((_B, _E), f32)),
    )(x_emb, se_w1, se_w2, emb_c_table, wq_T,
      qt.astype(f32).reshape(_B, 1), xt.reshape(_B, 1),
      qt_kc[:_C], graphs)

    operands = (
        ht, mf, adj0, adj1, res,
        aux, whtcat, gru_bhh.reshape(3 * _H, 1),
        pa_w, fsw2_aug, w2aug,
        ea_w_aug, wih_aug, ea_w.reshape(1, _C), wp_aug,
    )

    def full(a):
        nd = a.ndim
        return pl.BlockSpec(a.shape, lambda i, q, _n=nd: (0,) * _n)

    in_specs = [
        pl.BlockSpec((_BT, _C, _H), lambda i, q: (i, 0, 0)),
        pl.BlockSpec((_BT, _C), lambda i, q: (i, 0)),
        pl.BlockSpec((_BT, _C), lambda i, q: (i, 0)),
        pl.BlockSpec((_BT, _C), lambda i, q: (i, 0)),
        pl.BlockSpec((_BT, _E), lambda i, q: (i, 0)),
    ] + [full(a) for a in operands[5:]]

    grid_spec = pltpu.PrefetchScalarGridSpec(
        num_scalar_prefetch=1,
        grid=(_B // _BT,),
        in_specs=in_specs,
        out_specs=pl.BlockSpec((_BT, _C), lambda i, q: (i, 0)),
    )

    return pl.pallas_call(
        _gkt_kernel,
        grid_spec=grid_spec,
        out_shape=jax.ShapeDtypeStruct((_B, _C), f32),
    )(qt, *operands)


# speedup vs baseline: 8.5816x; 1.4977x over previous
import jax
import jax.numpy as jnp
from jax.experimental import pallas as pl
from jax.experimental.pallas import tpu as pltpu

_B, _C, _H, _E = 256, 1024, 32, 32
_D = _H + _E
_ET = 2
_BT = 32


def _sig(x):
    return 0.5 * jnp.tanh(0.5 * x) + 0.5


def _prologue_kernel(x_emb_ref, se_w1_ref, se_w2_ref, emb_c_ref,
                     wq_ref, qtf_ref, xt_ref, qt_kc_ref, graphs_ref,
                     aux_ref, mf_ref, adj0_ref, adj1_ref, res_ref):
    x = x_emb_ref[...]
    s_col = jnp.mean(x, axis=1, keepdims=True)
    s_row = jnp.transpose(s_col)
    h1 = jnp.maximum(jnp.dot(s_row, se_w1_ref[...],
                             preferred_element_type=jnp.float32), 0.0)
    scale_row = _sig(jnp.dot(h1, se_w2_ref[...],
                             preferred_element_type=jnp.float32))
    sc_x_emb = x * jnp.transpose(scale_row)

    e0 = jnp.transpose(emb_c_ref[0:1, :])
    e1 = jnp.transpose(emb_c_ref[1:2, :])
    ecols = jnp.concatenate([e0, e1 - e0, e1], axis=1)
    aux_ref[...] = jnp.dot(wq_ref[...], ecols,
                           preferred_element_type=jnp.float32)

    lane_iota = jax.lax.broadcasted_iota(
        jnp.int32, (_B, _C), 1).astype(jnp.float32)
    onehot = (lane_iota == qtf_ref[...]).astype(jnp.float32)
    mf = jnp.dot(onehot, qt_kc_ref[...],
                 preferred_element_type=jnp.float32)
    mf_ref[...] = mf

    denom = jnp.maximum(jnp.sum(mf, axis=1, keepdims=True), 1.0)
    mfn = mf * (1.0 / denom)
    adj0_ref[...] = jnp.dot(mfn, graphs_ref[0],
                            preferred_element_type=jnp.float32)
    adj1_ref[...] = jnp.dot(mfn, graphs_ref[1],
                            preferred_element_type=jnp.float32)

    res_ref[...] = jnp.dot(mf * xt_ref[...], sc_x_emb,
                           preferred_element_type=jnp.float32)


def _gkt_kernel(qt_s,
                ht_ref, mf_ref, adj0_ref, adj1_ref, res_ref,
                aux_ref, whtcat_ref, bhh_ref,
                pa_w_ref, fsw2aug_ref, w2aug_ref,
                eawaug_ref, wihaug_ref, eaw_row_ref, wpaug_ref,
                out_ref):
    i = pl.program_id(0)
    base = i * _BT

    qe0 = aux_ref[:, 0:1]
    qd = aux_ref[:, 1:2]
    qe1 = aux_ref[:, 2:3]
    lane_iota = jax.lax.broadcasted_iota(jnp.int32, (1, _C), 1)
    ones_c = jnp.ones((1, _C), jnp.float32)
    ones_bt = jnp.ones((1, _BT), jnp.float32)
    w_row = eaw_row_ref[...]

    ceq_rows = []
    htq_cols = []
    for j in range(_BT):
        q = qt_s[base + j]
        ceq = (lane_iota == q).astype(jnp.float32)
        ceq_rows.append(ceq)
        htq_cols.append(jnp.dot(ht_ref[j], jnp.transpose(ceq),
                                preferred_element_type=jnp.float32))
    scols_aug = jnp.concatenate(
        [jnp.concatenate(htq_cols, axis=1),
         jnp.transpose(res_ref[...]), ones_bt], axis=0)

    pa = jnp.dot(pa_w_ref[...], scols_aug,
                 preferred_element_type=jnp.float32)
    f1 = jnp.maximum(pa[:_H], 0.0)
    self_feat_all = jnp.dot(fsw2aug_ref[...],
                            jnp.concatenate([f1, ones_bt], axis=0),
                            preferred_element_type=jnp.float32)
    colones_fn = pa[_H:3 * _H] + qe0
    colceq_fn = pa[3 * _H:] - qe1
    cols_gru = jnp.concatenate(
        [bhh_ref[...], jnp.zeros((3 * _H, 2), jnp.float32)],
        axis=1)

    out_rows = []
    for j in range(_BT):
        ceq_row = ceq_rows[j]
        mf_row = mf_ref[j:j + 1, :]
        adj0_row = adj0_ref[j:j + 1, :]
        adj1_row = adj1_ref[j:j + 1, :]
        htT = ht_ref[j]

        cols_fn = jnp.concatenate(
            [colones_fn[:, j:j + 1], qd, colceq_fn[:, j:j + 1]],
            axis=1)
        w_aug = jnp.concatenate(
            [whtcat_ref[...],
             jnp.concatenate([cols_fn, cols_gru], axis=0)], axis=1)
        aug_in = jnp.concatenate(
            [htT, ones_c, mf_row, ceq_row], axis=0)
        big = jnp.dot(w_aug, aug_in,
                      preferred_element_type=jnp.float32)
        h1 = jnp.maximum(big[:2 * _H], 0.0)
        gh = big[2 * _H:]

        z = jnp.concatenate([h1[:_H] * adj0_row,
                             h1[_H:] * adj1_row,
                             adj0_row, adj1_row], axis=0)
        nf = jnp.dot(w2aug_ref[...], z,
                     preferred_element_type=jnp.float32)

        m_next = nf + ceq_row * (self_feat_all[:, j:j + 1] - nf)

        ea = jnp.dot(eawaug_ref[...],
                     jnp.concatenate([m_next, ones_c], axis=0),
                     preferred_element_type=jnp.float32)
        eg = _sig(ea[:_H])
        ag = jnp.tanh(ea[_H:])
        m2 = m_next - (w_row * eg) * m_next + w_row * ag

        gi = jnp.dot(wihaug_ref[...],
                     jnp.concatenate([m2, ones_c], axis=0),
                     preferred_element_type=jnp.float32)
        r = _sig(gi[:_H] + gh[:_H])
        zg = _sig(gi[_H:2 * _H] + gh[_H:2 * _H])
        n = jnp.tanh(gi[2 * _H:] + r * gh[2 * _H:])
        h_next = n + zg * (htT - n)

        yt_row = _sig(jnp.dot(wpaug_ref[...],
                              jnp.concatenate([h_next, ones_c], axis=0),
                              preferred_element_type=jnp.float32))
        out_rows.append(yt_row)

    out_ref[...] = jnp.concatenate(out_rows, axis=0)


def kernel(xt, qt, ht, qt_kc, emb_x_table, emb_c_table, se_w1, se_w2,
           fs_w1, fs_b1, fs_w2, fs_b2, fn_w1, fn_b1, fn_w2, fn_b2,
           ea_w, ea_we, ea_be, ea_wa, ea_ba,
           gru_wih, gru_bih, gru_whh, gru_bhh, wp, bp, graphs):
    f32 = jnp.float32
    x_emb = emb_x_table[:_C]

    wh_T = jnp.concatenate([fn_w1[0, _D:_D + _H].T,
                            fn_w1[1, _D:_D + _H].T], axis=0)
    wq_T = jnp.concatenate([fn_w1[0, _D + _H:].T,
                            fn_w1[1, _D + _H:].T], axis=0)
    wself_T = jnp.concatenate([fn_w1[0, :_D].T,
                               fn_w1[1, :_D].T], axis=0)
    b1cat = jnp.concatenate([fn_b1[0], fn_b1[1]]).reshape(2 * _H, 1)
    w2cat_T = jnp.concatenate([fn_w2[0].T, fn_w2[1].T], axis=1)
    b2_T = jnp.stack([fn_b2[0], fn_b2[1]], axis=1)
    eacat_T = jnp.concatenate([ea_we.T, ea_wa.T], axis=0)
    whtcat = jnp.concatenate([wh_T, gru_whh.T], axis=0)

    pa_w = jnp.concatenate([
        jnp.concatenate([fs_w1.T, fs_b1.reshape(_H, 1)], axis=1),
        jnp.concatenate([wself_T, b1cat], axis=1),
        jnp.concatenate([jnp.zeros((2 * _H, _H), f32), wq_T,
                         jnp.zeros((2 * _H, 1), f32)], axis=1),
    ], axis=0)
    fsw2_aug = jnp.concatenate([fs_w2.T, fs_b2.reshape(_H, 1)], axis=1)
    w2aug = jnp.concatenate([w2cat_T, b2_T], axis=1)
    ea_w_aug = jnp.concatenate(
        [eacat_T,
         jnp.concatenate([ea_be, ea_ba]).reshape(2 * _H, 1)], axis=1)
    wih_aug = jnp.concatenate([gru_wih.T, gru_bih.reshape(3 * _H, 1)],
                              axis=1)
    wp_aug = jnp.concatenate([wp.reshape(1, _H), bp.reshape(1, 1)],
                             axis=1)

    aux, mf, adj0, adj1, res = pl.pallas_call(
        _prologue_kernel,
        out_shape=(jax.ShapeDtypeStruct((2 * _H, 3), f32),
                   jax.ShapeDtypeStruct((_B, _C), f32),
                   jax.ShapeDtypeStruct((_B, _C), f32),
                   jax.ShapeDtypeStruct((_B, _C), f32),
                   jax.ShapeDtypeStruct((_B, _E), f32)),
    )(x_emb, se_w1, se_w2, emb_c_table, wq_T,
      qt.astype(f32).reshape(_B, 1), xt.reshape(_B, 1),
      qt_kc[:_C], graphs)

    operands = (
        jnp.transpose(ht, (0, 2, 1)), mf, adj0, adj1, res,
        aux, whtcat, gru_bhh.reshape(3 * _H, 1),
        pa_w, fsw2_aug, w2aug,
        ea_w_aug, wih_aug, ea_w.reshape(1, _C), wp_aug,
    )

    def full(a):
        nd = a.ndim
        return pl.BlockSpec(a.shape, lambda i, q, _n=nd: (0,) * _n)

    in_specs = [
        pl.BlockSpec((_BT, _H, _C), lambda i, q: (i, 0, 0)),
        pl.BlockSpec((_BT, _C), lambda i, q: (i, 0)),
        pl.BlockSpec((_BT, _C), lambda i, q: (i, 0)),
        pl.BlockSpec((_BT, _C), lambda i, q: (i, 0)),
        pl.BlockSpec((_BT, _E), lambda i, q: (i, 0)),
    ] + [full(a) for a in operands[5:]]

    grid_spec = pltpu.PrefetchScalarGridSpec(
        num_scalar_prefetch=1,
        grid=(_B // _BT,),
        in_specs=in_specs,
        out_specs=pl.BlockSpec((_BT, _C), lambda i, q: (i, 0)),
    )

    return pl.pallas_call(
        _gkt_kernel,
        grid_spec=grid_spec,
        out_shape=jax.ShapeDtypeStruct((_B, _C), f32),
    )(qt, *operands)


# speedup vs baseline: 8.6298x; 1.0056x over previous
import jax
import jax.numpy as jnp
from jax.experimental import pallas as pl
from jax.experimental.pallas import tpu as pltpu

_B, _C, _H, _E = 256, 1024, 32, 32
_D = _H + _E
_ET = 2
_BT = 64


def _sig(x):
    return 0.5 * jnp.tanh(0.5 * x) + 0.5


def _prologue_kernel(x_emb_ref, se_w1_ref, se_w2_ref, emb_c_ref,
                     wq_ref, qtf_ref, xt_ref, qt_kc_ref, graphs_ref,
                     aux_ref, mf_ref, adj0_ref, adj1_ref, res_ref):
    x = x_emb_ref[...]
    s_col = jnp.mean(x, axis=1, keepdims=True)
    s_row = jnp.transpose(s_col)
    h1 = jnp.maximum(jnp.dot(s_row, se_w1_ref[...],
                             preferred_element_type=jnp.float32), 0.0)
    scale_row = _sig(jnp.dot(h1, se_w2_ref[...],
                             preferred_element_type=jnp.float32))
    sc_x_emb = x * jnp.transpose(scale_row)

    e0 = jnp.transpose(emb_c_ref[0:1, :])
    e1 = jnp.transpose(emb_c_ref[1:2, :])
    ecols = jnp.concatenate([e0, e1 - e0, e1], axis=1)
    aux_ref[...] = jnp.dot(wq_ref[...], ecols,
                           preferred_element_type=jnp.float32)

    lane_iota = jax.lax.broadcasted_iota(
        jnp.int32, (_B, _C), 1).astype(jnp.float32)
    onehot = (lane_iota == qtf_ref[...]).astype(jnp.float32)
    mf = jnp.dot(onehot, qt_kc_ref[...],
                 preferred_element_type=jnp.float32)
    mf_ref[...] = mf

    denom = jnp.maximum(jnp.sum(mf, axis=1, keepdims=True), 1.0)
    mfn = mf * (1.0 / denom)
    adj0_ref[...] = jnp.dot(mfn, graphs_ref[0],
                            preferred_element_type=jnp.float32)
    adj1_ref[...] = jnp.dot(mfn, graphs_ref[1],
                            preferred_element_type=jnp.float32)

    res_ref[...] = jnp.dot(mf * xt_ref[...], sc_x_emb,
                           preferred_element_type=jnp.float32)


def _gkt_kernel(qt_s,
                ht_ref, mf_ref, adj0_ref, adj1_ref, res_ref,
                aux_ref, whtcat_ref, bhh_ref,
                pa_w_ref, fsw2aug_ref, w2aug_ref,
                eawaug_ref, wihaug_ref, eaw_row_ref, wpaug_ref,
                out_ref):
    i = pl.program_id(0)
    base = i * _BT

    qe0 = aux_ref[:, 0:1]
    qd = aux_ref[:, 1:2]
    qe1 = aux_ref[:, 2:3]
    lane_iota = jax.lax.broadcasted_iota(jnp.int32, (1, _C), 1)
    ones_c = jnp.ones((1, _C), jnp.float32)
    ones_bt = jnp.ones((1, _BT), jnp.float32)
    w_row = eaw_row_ref[...]

    ceq_rows = []
    htq_cols = []
    for j in range(_BT):
        q = qt_s[base + j]
        ceq = (lane_iota == q).astype(jnp.float32)
        ceq_rows.append(ceq)
        htq_cols.append(jnp.dot(ht_ref[j], jnp.transpose(ceq),
                                preferred_element_type=jnp.float32))
    scols_aug = jnp.concatenate(
        [jnp.concatenate(htq_cols, axis=1),
         jnp.transpose(res_ref[...]), ones_bt], axis=0)

    pa = jnp.dot(pa_w_ref[...], scols_aug,
                 preferred_element_type=jnp.float32)
    f1 = jnp.maximum(pa[:_H], 0.0)
    self_feat_all = jnp.dot(fsw2aug_ref[...],
                            jnp.concatenate([f1, ones_bt], axis=0),
                            preferred_element_type=jnp.float32)
    colones_fn = pa[_H:3 * _H] + qe0
    colceq_fn = pa[3 * _H:] - qe1
    cols_gru = jnp.concatenate(
        [bhh_ref[...], jnp.zeros((3 * _H, 2), jnp.float32)],
        axis=1)

    out_rows = []
    for j in range(_BT):
        ceq_row = ceq_rows[j]
        mf_row = mf_ref[j:j + 1, :]
        adj0_row = adj0_ref[j:j + 1, :]
        adj1_row = adj1_ref[j:j + 1, :]
        htT = ht_ref[j]

        cols_fn = jnp.concatenate(
            [colones_fn[:, j:j + 1], qd, colceq_fn[:, j:j + 1]],
            axis=1)
        w_aug = jnp.concatenate(
            [whtcat_ref[...],
             jnp.concatenate([cols_fn, cols_gru], axis=0)], axis=1)
        aug_in = jnp.concatenate(
            [htT, ones_c, mf_row, ceq_row], axis=0)
        big = jnp.dot(w_aug, aug_in,
                      preferred_element_type=jnp.float32)
        h1 = jnp.maximum(big[:2 * _H], 0.0)
        gh = big[2 * _H:]

        z = jnp.concatenate([h1[:_H] * adj0_row,
                             h1[_H:] * adj1_row,
                             adj0_row, adj1_row], axis=0)
        nf = jnp.dot(w2aug_ref[...], z,
                     preferred_element_type=jnp.float32)

        m_next = nf + ceq_row * (self_feat_all[:, j:j + 1] - nf)

        ea = jnp.dot(eawaug_ref[...],
                     jnp.concatenate([m_next, ones_c], axis=0),
                     preferred_element_type=jnp.float32)
        eg = _sig(ea[:_H])
        ag = jnp.tanh(ea[_H:])
        m2 = m_next - (w_row * eg) * m_next + w_row * ag

        gi = jnp.dot(wihaug_ref[...],
                     jnp.concatenate([m2, ones_c], axis=0),
                     preferred_element_type=jnp.float32)
        r = _sig(gi[:_H] + gh[:_H])
        zg = _sig(gi[_H:2 * _H] + gh[_H:2 * _H])
        n = jnp.tanh(gi[2 * _H:] + r * gh[2 * _H:])
        h_next = n + zg * (htT - n)

        yt_row = _sig(jnp.dot(wpaug_ref[...],
                              jnp.concatenate([h_next, ones_c], axis=0),
                              preferred_element_type=jnp.float32))
        out_rows.append(yt_row)

    out_ref[...] = jnp.concatenate(out_rows, axis=0)


def kernel(xt, qt, ht, qt_kc, emb_x_table, emb_c_table, se_w1, se_w2,
           fs_w1, fs_b1, fs_w2, fs_b2, fn_w1, fn_b1, fn_w2, fn_b2,
           ea_w, ea_we, ea_be, ea_wa, ea_ba,
           gru_wih, gru_bih, gru_whh, gru_bhh, wp, bp, graphs):
    f32 = jnp.float32
    x_emb = emb_x_table[:_C]

    wh_T = jnp.concatenate([fn_w1[0, _D:_D + _H].T,
                            fn_w1[1, _D:_D + _H].T], axis=0)
    wq_T = jnp.concatenate([fn_w1[0, _D + _H:].T,
                            fn_w1[1, _D + _H:].T], axis=0)
    wself_T = jnp.concatenate([fn_w1[0, :_D].T,
                               fn_w1[1, :_D].T], axis=0)
    b1cat = jnp.concatenate([fn_b1[0], fn_b1[1]]).reshape(2 * _H, 1)
    w2cat_T = jnp.concatenate([fn_w2[0].T, fn_w2[1].T], axis=1)
    b2_T = jnp.stack([fn_b2[0], fn_b2[1]], axis=1)
    eacat_T = jnp.concatenate([ea_we.T, ea_wa.T], axis=0)
    whtcat = jnp.concatenate([wh_T, gru_whh.T], axis=0)

    pa_w = jnp.concatenate([
        jnp.concatenate([fs_w1.T, fs_b1.reshape(_H, 1)], axis=1),
        jnp.concatenate([wself_T, b1cat], axis=1),
        jnp.concatenate([jnp.zeros((2 * _H, _H), f32), wq_T,
                         jnp.zeros((2 * _H, 1), f32)], axis=1),
    ], axis=0)
    fsw2_aug = jnp.concatenate([fs_w2.T, fs_b2.reshape(_H, 1)], axis=1)
    w2aug = jnp.concatenate([w2cat_T, b2_T], axis=1)
    ea_w_aug = jnp.concatenate(
        [eacat_T,
         jnp.concatenate([ea_be, ea_ba]).reshape(2 * _H, 1)], axis=1)
    wih_aug = jnp.concatenate([gru_wih.T, gru_bih.reshape(3 * _H, 1)],
                              axis=1)
    wp_aug = jnp.concatenate([wp.reshape(1, _H), bp.reshape(1, 1)],
                             axis=1)

    aux, mf, adj0, adj1, res = pl.pallas_call(
        _prologue_kernel,
        out_shape=(jax.ShapeDtypeStruct((2 * _H, 3), f32),
                   jax.ShapeDtypeStruct((_B, _C), f32),
                   jax.ShapeDtypeStruct((_B, _C), f32),
                   jax.ShapeDtypeStruct((_B, _C), f32),
                   jax.ShapeDtypeStruct((_B, _E), f32)),
    )(x_emb, se_w1, se_w2, emb_c_table, wq_T,
      qt.astype(f32).reshape(_B, 1), xt.reshape(_B, 1),
      qt_kc[:_C], graphs)

    operands = (
        jnp.transpose(ht, (0, 2, 1)), mf, adj0, adj1, res,
        aux, whtcat, gru_bhh.reshape(3 * _H, 1),
        pa_w, fsw2_aug, w2aug,
        ea_w_aug, wih_aug, ea_w.reshape(1, _C), wp_aug,
    )

    def full(a):
        nd = a.ndim
        return pl.BlockSpec(a.shape, lambda i, q, _n=nd: (0,) * _n)

    in_specs = [
        pl.BlockSpec((_BT, _H, _C), lambda i, q: (i, 0, 0)),
        pl.BlockSpec((_BT, _C), lambda i, q: (i, 0)),
        pl.BlockSpec((_BT, _C), lambda i, q: (i, 0)),
        pl.BlockSpec((_BT, _C), lambda i, q: (i, 0)),
        pl.BlockSpec((_BT, _E), lambda i, q: (i, 0)),
    ] + [full(a) for a in operands[5:]]

    grid_spec = pltpu.PrefetchScalarGridSpec(
        num_scalar_prefetch=1,
        grid=(_B // _BT,),
        in_specs=in_specs,
        out_specs=pl.BlockSpec((_BT, _C), lambda i, q: (i, 0)),
    )

    return pl.pallas_call(
        _gkt_kernel,
        grid_spec=grid_spec,
        out_shape=jax.ShapeDtypeStruct((_B, _C), f32),
    )(qt, *operands)


# speedup vs baseline: 15.8555x; 1.8373x over previous
import jax
import jax.numpy as jnp
from jax.experimental import pallas as pl
from jax.experimental.pallas import tpu as pltpu

_B, _C, _H, _E = 256, 1024, 32, 32
_D = _H + _E
_ET = 2
_BT = 32
_G = 8


def _sig(x):
    return 0.5 * jnp.tanh(0.5 * x) + 0.5


def _prologue_kernel(x_emb_ref, se_w1_ref, se_w2_ref, emb_c_ref,
                     wq_ref, qtf_ref, xt_ref, qt_kc_ref, graphs_ref,
                     aux_ref, mf_ref, adj0_ref, adj1_ref, res_ref):
    x = x_emb_ref[...]
    s_col = jnp.mean(x, axis=1, keepdims=True)
    s_row = jnp.transpose(s_col)
    h1 = jnp.maximum(jnp.dot(s_row, se_w1_ref[...],
                             preferred_element_type=jnp.float32), 0.0)
    scale_row = _sig(jnp.dot(h1, se_w2_ref[...],
                             preferred_element_type=jnp.float32))
    sc_x_emb = x * jnp.transpose(scale_row)

    e0 = jnp.transpose(emb_c_ref[0:1, :])
    e1 = jnp.transpose(emb_c_ref[1:2, :])
    ecols = jnp.concatenate([e0, e1 - e0, e1], axis=1)
    aux_ref[...] = jnp.dot(wq_ref[...], ecols,
                           preferred_element_type=jnp.float32)

    lane_iota = jax.lax.broadcasted_iota(
        jnp.int32, (_B, _C), 1).astype(jnp.float32)
    onehot = (lane_iota == qtf_ref[...]).astype(jnp.float32)
    mf = jnp.dot(onehot, qt_kc_ref[...],
                 preferred_element_type=jnp.float32)
    mf_ref[...] = mf

    denom = jnp.maximum(jnp.sum(mf, axis=1, keepdims=True), 1.0)
    mfn = mf * (1.0 / denom)
    adj0_ref[...] = jnp.dot(mfn, graphs_ref[0],
                            preferred_element_type=jnp.float32)
    adj1_ref[...] = jnp.dot(mfn, graphs_ref[1],
                            preferred_element_type=jnp.float32)

    res_ref[...] = jnp.dot(mf * xt_ref[...], sc_x_emb,
                           preferred_element_type=jnp.float32)


def _gkt_kernel(qt_s,
                ht_ref, mf_ref, adj0_ref, adj1_ref, res_ref,
                aux_ref, whtcat_ref, bhh_ref,
                pa_w_ref, fsw2aug_ref, w2aug_ref,
                eawaug_ref, wihaug_ref, eaw_row_ref, wpaug_ref,
                out_ref):
    i = pl.program_id(0)
    base = i * _BT

    qe0 = aux_ref[:, 0:1]
    qd = aux_ref[:, 1:2]
    qe1 = aux_ref[:, 2:3]
    lane_iota = jax.lax.broadcasted_iota(jnp.int32, (1, _C), 1)
    ones_c = jnp.ones((1, _C), jnp.float32)
    ones_bt = jnp.ones((1, _BT), jnp.float32)
    w_row = eaw_row_ref[...]

    ceq_rows = []
    htq_cols = []
    for j in range(_BT):
        q = qt_s[base + j]
        ceq = (lane_iota == q).astype(jnp.float32)
        ceq_rows.append(ceq)
        htq_cols.append(jnp.dot(ht_ref[j], jnp.transpose(ceq),
                                preferred_element_type=jnp.float32))
    scols_aug = jnp.concatenate(
        [jnp.concatenate(htq_cols, axis=1),
         jnp.transpose(res_ref[...]), ones_bt], axis=0)

    pa = jnp.dot(pa_w_ref[...], scols_aug,
                 preferred_element_type=jnp.float32)
    f1 = jnp.maximum(pa[:_H], 0.0)
    self_feat_all = jnp.dot(fsw2aug_ref[...],
                            jnp.concatenate([f1, ones_bt], axis=0),
                            preferred_element_type=jnp.float32)
    colones_fn = pa[_H:3 * _H] + qe0
    colceq_fn = pa[3 * _H:] - qe1
    cols_gru = jnp.concatenate(
        [bhh_ref[...], jnp.zeros((3 * _H, 2), jnp.float32)],
        axis=1)

    ones_gc = jnp.ones((1, _G * _C), jnp.float32)
    w_all = jnp.concatenate([w_row] * _G, axis=1)
    blk_iota = jax.lax.broadcasted_iota(
        jnp.int32, (_G, _G * _C), 1) // _C
    sub_iota = jax.lax.broadcasted_iota(
        jnp.int32, (_G, _G * _C), 0)
    blk_sel = (sub_iota == blk_iota).astype(jnp.float32)

    out_groups = []
    for g in range(_BT // _G):
        js = range(g * _G, (g + 1) * _G)
        z_parts, gh_parts = [], []
        for j in js:
            ceq_row = ceq_rows[j]
            mf_row = mf_ref[j:j + 1, :]
            adj0_row = adj0_ref[j:j + 1, :]
            adj1_row = adj1_ref[j:j + 1, :]

            cols_fn = jnp.concatenate(
                [colones_fn[:, j:j + 1], qd, colceq_fn[:, j:j + 1]],
                axis=1)
            w_aug = jnp.concatenate(
                [whtcat_ref[...],
                 jnp.concatenate([cols_fn, cols_gru], axis=0)], axis=1)
            aug_in = jnp.concatenate(
                [ht_ref[j], ones_c, mf_row, ceq_row], axis=0)
            big = jnp.dot(w_aug, aug_in,
                          preferred_element_type=jnp.float32)
            h1 = jnp.maximum(big[:2 * _H], 0.0)
            gh_parts.append(big[2 * _H:])

            z_parts.append(jnp.concatenate(
                [h1[:_H] * adj0_row, h1[_H:] * adj1_row,
                 adj0_row, adj1_row], axis=0))

        z_all = jnp.concatenate(z_parts, axis=1)
        gh_all = jnp.concatenate(gh_parts, axis=1)
        ceq_all = jnp.concatenate(
            [ceq_rows[j] for j in js], axis=1)
        htT_all = jnp.concatenate(
            [ht_ref[j] for j in js], axis=1)

        nf_all = jnp.dot(w2aug_ref[...], z_all,
                         preferred_element_type=jnp.float32)

        t_g = blk_sel * ceq_all
        outer_all = jnp.dot(self_feat_all[:, g * _G:(g + 1) * _G], t_g,
                            preferred_element_type=jnp.float32)
        m_next = nf_all * (1.0 - ceq_all) + outer_all

        ea = jnp.dot(eawaug_ref[...],
                     jnp.concatenate([m_next, ones_gc], axis=0),
                     preferred_element_type=jnp.float32)
        eg = _sig(ea[:_H])
        ag = jnp.tanh(ea[_H:])
        m2 = m_next - (w_all * eg) * m_next + w_all * ag

        gi = jnp.dot(wihaug_ref[...],
                     jnp.concatenate([m2, ones_gc], axis=0),
                     preferred_element_type=jnp.float32)
        r = _sig(gi[:_H] + gh_all[:_H])
        zg = _sig(gi[_H:2 * _H] + gh_all[_H:2 * _H])
        n = jnp.tanh(gi[2 * _H:] + r * gh_all[2 * _H:])
        h_next = n + zg * (htT_all - n)

        out_groups.append(_sig(jnp.dot(
            wpaug_ref[...],
            jnp.concatenate([h_next, ones_gc], axis=0),
            preferred_element_type=jnp.float32)))

    out_ref[0] = jnp.concatenate(out_groups, axis=1)


def kernel(xt, qt, ht, qt_kc, emb_x_table, emb_c_table, se_w1, se_w2,
           fs_w1, fs_b1, fs_w2, fs_b2, fn_w1, fn_b1, fn_w2, fn_b2,
           ea_w, ea_we, ea_be, ea_wa, ea_ba,
           gru_wih, gru_bih, gru_whh, gru_bhh, wp, bp, graphs):
    f32 = jnp.float32
    x_emb = emb_x_table[:_C]

    wh_T = jnp.concatenate([fn_w1[0, _D:_D + _H].T,
                            fn_w1[1, _D:_D + _H].T], axis=0)
    wq_T = jnp.concatenate([fn_w1[0, _D + _H:].T,
                            fn_w1[1, _D + _H:].T], axis=0)
    wself_T = jnp.concatenate([fn_w1[0, :_D].T,
                               fn_w1[1, :_D].T], axis=0)
    b1cat = jnp.concatenate([fn_b1[0], fn_b1[1]]).reshape(2 * _H, 1)
    w2cat_T = jnp.concatenate([fn_w2[0].T, fn_w2[1].T], axis=1)
    b2_T = jnp.stack([fn_b2[0], fn_b2[1]], axis=1)
    eacat_T = jnp.concatenate([ea_we.T, ea_wa.T], axis=0)
    whtcat = jnp.concatenate([wh_T, gru_whh.T], axis=0)

    pa_w = jnp.concatenate([
        jnp.concatenate([fs_w1.T, fs_b1.reshape(_H, 1)], axis=1),
        jnp.concatenate([wself_T, b1cat], axis=1),
        jnp.concatenate([jnp.zeros((2 * _H, _H), f32), wq_T,
                         jnp.zeros((2 * _H, 1), f32)], axis=1),
    ], axis=0)
    fsw2_aug = jnp.concatenate([fs_w2.T, fs_b2.reshape(_H, 1)], axis=1)
    w2aug = jnp.concatenate([w2cat_T, b2_T], axis=1)
    ea_w_aug = jnp.concatenate(
        [eacat_T,
         jnp.concatenate([ea_be, ea_ba]).reshape(2 * _H, 1)], axis=1)
    wih_aug = jnp.concatenate([gru_wih.T, gru_bih.reshape(3 * _H, 1)],
                              axis=1)
    wp_aug = jnp.concatenate([wp.reshape(1, _H), bp.reshape(1, 1)],
                             axis=1)

    aux, mf, adj0, adj1, res = pl.pallas_call(
        _prologue_kernel,
        out_shape=(jax.ShapeDtypeStruct((2 * _H, 3), f32),
                   jax.ShapeDtypeStruct((_B, _C), f32),
                   jax.ShapeDtypeStruct((_B, _C), f32),
                   jax.ShapeDtypeStruct((_B, _C), f32),
                   jax.ShapeDtypeStruct((_B, _E), f32)),
    )(x_emb, se_w1, se_w2, emb_c_table, wq_T,
      qt.astype(f32).reshape(_B, 1), xt.reshape(_B, 1),
      qt_kc[:_C], graphs)

    operands = (
        jnp.transpose(ht, (0, 2, 1)), mf, adj0, adj1, res,
        aux, whtcat, gru_bhh.reshape(3 * _H, 1),
        pa_w, fsw2_aug, w2aug,
        ea_w_aug, wih_aug, ea_w.reshape(1, _C), wp_aug,
    )

    def full(a):
        nd = a.ndim
        return pl.BlockSpec(a.shape, lambda i, q, _n=nd: (0,) * _n)

    in_specs = [
        pl.BlockSpec((_BT, _H, _C), lambda i, q: (i, 0, 0)),
        pl.BlockSpec((_BT, _C), lambda i, q: (i, 0)),
        pl.BlockSpec((_BT, _C), lambda i, q: (i, 0)),
        pl.BlockSpec((_BT, _C), lambda i, q: (i, 0)),
        pl.BlockSpec((_BT, _E), lambda i, q: (i, 0)),
    ] + [full(a) for a in operands[5:]]

    grid_spec = pltpu.PrefetchScalarGridSpec(
        num_scalar_prefetch=1,
        grid=(_B // _BT,),
        in_specs=in_specs,
        out_specs=pl.BlockSpec((1, 1, _BT * _C), lambda i, q: (i, 0, 0)),
    )

    yt = pl.pallas_call(
        _gkt_kernel,
        grid_spec=grid_spec,
        out_shape=jax.ShapeDtypeStruct((_B // _BT, 1, _BT * _C), f32),
    )(qt, *operands)
    return yt.reshape(_B, _C)
